# baseline (speedup 1.0000x reference)
import jax
import jax.numpy as jnp
from jax.experimental import pallas as pl
from jax.experimental.pallas import tpu as pltpu

EPS = 1e-7
MAX_NORM = 1.0 - 1e-5


def _logmap0_body(x_ref, o_ref):
    x = x_ref[...]
    norm = jnp.clip(jnp.sqrt(jnp.sum(x * x, axis=-1, keepdims=True)), EPS, None)
    z = jnp.clip(norm, None, MAX_NORM)
    atanh = 0.5 * jnp.log((1.0 + z) / (1.0 - z))
    o_ref[...] = atanh * x / norm


def _strip(blk, xt):
    acc = jnp.dot(blk, xt, preferred_element_type=jnp.float32)
    cnt = jnp.sum(blk, axis=1, keepdims=True)
    agg = acc / cnt
    norm = jnp.clip(jnp.sqrt(jnp.sum(agg * agg, axis=-1, keepdims=True)), EPS, None)
    res = jnp.tanh(norm) * agg / norm
    norm2 = jnp.clip(jnp.sqrt(jnp.sum(res * res, axis=-1, keepdims=True)), EPS, None)
    return res * jnp.minimum(1.0, MAX_NORM / norm2)


def _agg_body(adj0_ref, adj1_ref, xt_ref, o0_ref, o1_ref):
    xt = xt_ref[...]
    o0_ref[...] = _strip(adj0_ref[...], xt)
    o1_ref[...] = _strip(adj1_ref[...], xt)


def kernel(x, adj):
    N, D = x.shape
    xt = pl.pallas_call(
        _logmap0_body,
        grid=(5,),
        in_specs=[pl.BlockSpec((N // 5, D), lambda i: (i, 0))],
        out_specs=pl.BlockSpec((N // 5, D), lambda i: (i, 0)),
        out_shape=jax.ShapeDtypeStruct((N, D), jnp.float32),
    )(x)

    BM = 200
    G = N // (2 * BM)
    o0, o1 = pl.pallas_call(
        _agg_body,
        grid=(G,),
        in_specs=[
            pl.BlockSpec((BM, N), lambda i: (2 * i, 0)),
            pl.BlockSpec((BM, N), lambda i: (2 * i + 1, 0)),
            pl.BlockSpec((N, D), lambda i: (0, 0)),
        ],
        out_specs=[
            pl.BlockSpec((BM, D), lambda i: (i, 0)),
            pl.BlockSpec((BM, D), lambda i: (i, 0)),
        ],
        out_shape=[
            jax.ShapeDtypeStruct((N // 2, D), jnp.float32),
            jax.ShapeDtypeStruct((N // 2, D), jnp.float32),
        ],
        compiler_params=pltpu.CompilerParams(
            dimension_semantics=("parallel",),
        ),
    )(adj, adj, xt)
    out = jnp.stack([o0.reshape(G, BM, D), o1.reshape(G, BM, D)], axis=1)
    return out.reshape(N, D)


# speedup vs baseline: 1.0880x; 1.0880x over previous
import jax
import jax.numpy as jnp
from jax.experimental import pallas as pl
from jax.experimental.pallas import tpu as pltpu

EPS = 1e-7
MAX_NORM = 1.0 - 1e-5


def _fused_body(x_ref, adj_ref, o_ref, xt_ref):
    @pl.when(pl.program_id(0) == 0)
    def _():
        xx = x_ref[...]
        norm = jnp.clip(jnp.sqrt(jnp.sum(xx * xx, axis=-1, keepdims=True)), EPS, None)
        z = jnp.clip(norm, None, MAX_NORM)
        atanh = 0.5 * jnp.log((1.0 + z) / (1.0 - z))
        xt_ref[...] = atanh * xx / norm

    blk = adj_ref[...]
    acc = jnp.dot(blk, xt_ref[...], preferred_element_type=jnp.float32)
    cnt = jnp.sum(blk, axis=1, keepdims=True)
    agg = acc / cnt
    norm = jnp.clip(jnp.sqrt(jnp.sum(agg * agg, axis=-1, keepdims=True)), EPS, None)
    res = jnp.tanh(norm) * agg / norm
    norm2 = jnp.clip(jnp.sqrt(jnp.sum(res * res, axis=-1, keepdims=True)), EPS, None)
    o_ref[...] = res * jnp.minimum(1.0, MAX_NORM / norm2)


def kernel(x, adj):
    N, D = x.shape
    BM = 400
    return pl.pallas_call(
        _fused_body,
        grid=(N // BM,),
        in_specs=[
            pl.BlockSpec((N, D), lambda i: (0, 0)),
            pl.BlockSpec((BM, N), lambda i: (i, 0)),
        ],
        out_specs=pl.BlockSpec((BM, D), lambda i: (i, 0)),
        out_shape=jax.ShapeDtypeStruct((N, D), jnp.float32),
        scratch_shapes=[pltpu.VMEM((N, D), jnp.float32)],
        compiler_params=pltpu.CompilerParams(
            dimension_semantics=("arbitrary",),
        ),
    )(x, adj)
